# Initial kernel scaffold; baseline (speedup 1.0000x reference)
import jax
import jax.numpy as jnp
from jax.experimental import pallas as pl


def kernel(params, g_edge_index, t_edge_index, s_edge_index):
    raise NotImplementedError("write your pallas kernel here")


# speedup vs baseline: 2.1665x; 2.1665x over previous
import functools

import jax
import jax.numpy as jnp
from jax import lax
from jax.experimental import pallas as pl
from jax.experimental.pallas import tpu as pltpu
from jax.experimental.pallas import tpu_sc as plsc

N_MASHUP = 5000
N_NODES = 10000
EMB = 128
LOCAL = 64
N_EDGES = 320000
TAU = 0.6

NC = 2
NS = 16
NW = NC * NS
CH = 128
EPT = 10240
KCH = EPT // CH
EPAD = NW * EPT
NNP = 10240
RPS = NNP // NS
DEG_W = 128


@functools.lru_cache(maxsize=None)
def _make_sc_agg(d):
  mesh = plsc.VectorSubcoreMesh(core_axis_name="c", subcore_axis_name="s")
  out_type = [jax.ShapeDtypeStruct((NC, NNP, d), jnp.float32)]
  scratch = [
      pltpu.VMEM((KCH, CH), jnp.int32),
      pltpu.VMEM((KCH, CH), jnp.int32),
      pltpu.VMEM((CH, d), jnp.float32),
      pltpu.VMEM_SHARED((NNP, d), jnp.float32),
      pltpu.SemaphoreType.DMA,
  ]

  def body(x_hbm, src_hbm, dst_hbm, zeros_hbm,
           out_hbm, src_v, dst_v, rows_v, acc_sh, gsem):
    ci = lax.axis_index("c")
    si = lax.axis_index("s")
    wid = si * NC + ci
    pltpu.sync_copy(src_hbm.at[wid], src_v)
    pltpu.sync_copy(dst_hbm.at[wid], dst_v)
    pltpu.sync_copy(zeros_hbm.at[pl.ds(si * RPS, RPS)],
                    acc_sh.at[pl.ds(si * RPS, RPS)])
    plsc.subcore_barrier()

    @pl.loop(0, KCH)
    def _(j):
      pltpu.async_copy(x_hbm.at[src_v.at[j]], rows_v, gsem).wait()
      pltpu.sync_copy(rows_v, acc_sh.at[dst_v.at[j]], add=True)

    plsc.subcore_barrier()
    pltpu.sync_copy(acc_sh.at[pl.ds(si * RPS, RPS)],
                    out_hbm.at[ci, pl.ds(si * RPS, RPS)])

  return pl.kernel(body, out_type=out_type, mesh=mesh,
                   scratch_types=scratch, name=f"sc_agg_d{d}")


@functools.lru_cache(maxsize=None)
def _make_sc_deg():
  mesh = plsc.VectorSubcoreMesh(core_axis_name="c", subcore_axis_name="s")
  out_type = [jax.ShapeDtypeStruct((NC, NNP, DEG_W), jnp.float32)]
  scratch = [
      pltpu.VMEM((KCH, CH), jnp.int32),
      pltpu.VMEM((CH, DEG_W), jnp.float32),
      pltpu.VMEM_SHARED((NNP, DEG_W), jnp.float32),
  ]

  def body(dst_hbm, zdeg_hbm, ones_hbm, deg_hbm, dst_v, ones_v, deg_sh):
    ci = lax.axis_index("c")
    si = lax.axis_index("s")
    wid = si * NC + ci
    pltpu.sync_copy(dst_hbm.at[wid], dst_v)
    pltpu.sync_copy(ones_hbm, ones_v)
    pltpu.sync_copy(zdeg_hbm.at[pl.ds(si * RPS, RPS)],
                    deg_sh.at[pl.ds(si * RPS, RPS)])
    plsc.subcore_barrier()

    @pl.loop(0, KCH)
    def _(j):
      pltpu.sync_copy(ones_v, deg_sh.at[dst_v.at[j]], add=True)

    plsc.subcore_barrier()
    pltpu.sync_copy(deg_sh.at[pl.ds(si * RPS, RPS)],
                    deg_hbm.at[ci, pl.ds(si * RPS, RPS)])

  return pl.kernel(body, out_type=out_type, mesh=mesh,
                   scratch_types=scratch, name="sc_deg")


def _prep_edges(edge_index):
  src = edge_index[0]
  dst = edge_index[1]
  src = jnp.concatenate([src, jnp.zeros((EPAD - N_EDGES,), jnp.int32)])
  dst = jnp.concatenate(
      [dst, jnp.full((EPAD - N_EDGES,), N_NODES, jnp.int32)])
  return src.reshape(NW, KCH, CH), dst.reshape(NW, KCH, CH)


def _sc_agg(x, src3, dst3):
  d = x.shape[1]
  zeros = jnp.zeros((NNP, d), jnp.float32)
  return _make_sc_agg(d)(x, src3, dst3, zeros)[0]


def _sc_deg(dst3):
  zdeg = jnp.zeros((NNP, DEG_W), jnp.float32)
  ones = jnp.ones((CH, DEG_W), jnp.float32)
  return _make_sc_deg()(dst3, zdeg, ones)[0]


def _sage_dense(x, agg2, deg2, wsn, b, relu):
  n, din = x.shape
  dout = wsn.shape[1]
  blk = 2000
  grid = n // blk

  def body(x_ref, a_ref, d_ref, w_ref, b_ref, o_ref):
    agg = a_ref[0] + a_ref[1]
    deg = d_ref[0, :, 0:1] + d_ref[1, :, 0:1]
    hn = agg / jnp.maximum(deg, 1.0)
    xa = jnp.concatenate([x_ref[...], hn], axis=1)
    o = jnp.dot(xa, w_ref[...], preferred_element_type=jnp.float32)
    o = o + b_ref[...]
    if relu:
      o = jnp.maximum(o, 0.0)
    o_ref[...] = o

  return pl.pallas_call(
      body,
      grid=(grid,),
      in_specs=[
          pl.BlockSpec((blk, din), lambda i: (i, 0)),
          pl.BlockSpec((NC, blk, din), lambda i: (0, i, 0)),
          pl.BlockSpec((NC, blk, DEG_W), lambda i: (0, i, 0)),
          pl.BlockSpec((2 * din, dout), lambda i: (0, 0)),
          pl.BlockSpec((1, dout), lambda i: (0, 0)),
      ],
      out_specs=pl.BlockSpec((blk, dout), lambda i: (i, 0)),
      out_shape=jax.ShapeDtypeStruct((n, dout), jnp.float32),
  )(x, agg2, deg2, wsn, b)


def _graphsage(x, src3, dst3, p):
  dh = p['W1s'].shape[1]
  deg = _sc_deg(dst3)
  w1 = jnp.concatenate([p['W1s'], p['W1n']], axis=0)
  b1 = p['b1'].reshape(1, dh)
  if dh < EMB:
    w1 = jnp.pad(w1, ((0, 0), (0, EMB - dh)))
    b1 = jnp.pad(b1, ((0, 0), (0, EMB - dh)))
  agg1 = _sc_agg(x, src3, dst3)
  h = _sage_dense(x, agg1, deg, w1, b1, True)
  agg2 = _sc_agg(h, src3, dst3)
  w2s, w2n = p['W2s'], p['W2n']
  if dh < EMB:
    w2s = jnp.pad(w2s, ((0, EMB - dh), (0, 0)))
    w2n = jnp.pad(w2n, ((0, EMB - dh), (0, 0)))
  w2 = jnp.concatenate([w2s, w2n], axis=0)
  return _sage_dense(h, agg2, deg, w2, p['b2'].reshape(1, dh), False)


def _mlp_norm(a, fcp):
  n, d = a.shape
  blk = 1000
  grid = n // blk

  def body(a_ref, w1, b1, w2, b2, o_ref):
    h = jnp.dot(a_ref[...], w1[...], preferred_element_type=jnp.float32)
    h = jnp.maximum(h + b1[...], 0.0)
    z = jnp.dot(h, w2[...], preferred_element_type=jnp.float32) + b2[...]
    nrm = jnp.sqrt(jnp.sum(z * z, axis=1, keepdims=True))
    o_ref[...] = z / jnp.maximum(nrm, 1e-12)

  return pl.pallas_call(
      body,
      grid=(grid,),
      in_specs=[
          pl.BlockSpec((blk, d), lambda i: (i, 0)),
          pl.BlockSpec((d, d), lambda i: (0, 0)),
          pl.BlockSpec((1, d), lambda i: (0, 0)),
          pl.BlockSpec((d, d), lambda i: (0, 0)),
          pl.BlockSpec((1, d), lambda i: (0, 0)),
      ],
      out_specs=pl.BlockSpec((blk, d), lambda i: (i, 0)),
      out_shape=jax.ShapeDtypeStruct((n, d), jnp.float32),
  )(a, fcp['W1'], fcp['b1'].reshape(1, d), fcp['W2'], fcp['b2'].reshape(1, d))


def _contrast_sums(an, bn):
  n, d = an.shape
  blk = 200
  grid = n // blk
  inv_tau = 1.0 / TAU

  def body(ai_ref, bi_ref, an_ref, bn_ref,
           ra_ref, rb_ref, sr_ref, sc_ref, dg_ref, da_ref, db_ref):
    i = pl.program_id(0)
    ai = ai_ref[...]
    bi = bi_ref[...]
    dn = (((1,), (1,)), ((), ()))
    eaa = jnp.exp(lax.dot_general(ai, an_ref[...], dn,
                                  preferred_element_type=jnp.float32)
                  * inv_tau)
    ra_ref[...] = jnp.sum(eaa, axis=1, keepdims=True)
    ebb = jnp.exp(lax.dot_general(bi, bn_ref[...], dn,
                                  preferred_element_type=jnp.float32)
                  * inv_tau)
    rb_ref[...] = jnp.sum(ebb, axis=1, keepdims=True)
    eab = jnp.exp(lax.dot_general(ai, bn_ref[...], dn,
                                  preferred_element_type=jnp.float32)
                  * inv_tau)
    sr_ref[...] = jnp.sum(eab, axis=1, keepdims=True)
    colpart = jnp.sum(eab, axis=0, keepdims=True)

    @pl.when(i == 0)
    def _():
      sc_ref[...] = colpart

    @pl.when(i > 0)
    def _():
      sc_ref[...] += colpart

    dg_ref[...] = jnp.exp(jnp.sum(ai * bi, axis=1, keepdims=True) * inv_tau)
    da_ref[...] = jnp.exp(jnp.sum(ai * ai, axis=1, keepdims=True) * inv_tau)
    db_ref[...] = jnp.exp(jnp.sum(bi * bi, axis=1, keepdims=True) * inv_tau)

  col = pl.BlockSpec((blk, 1), lambda i: (i, 0))
  outs = [
      jax.ShapeDtypeStruct((n, 1), jnp.float32),
      jax.ShapeDtypeStruct((n, 1), jnp.float32),
      jax.ShapeDtypeStruct((n, 1), jnp.float32),
      jax.ShapeDtypeStruct((1, n), jnp.float32),
      jax.ShapeDtypeStruct((n, 1), jnp.float32),
      jax.ShapeDtypeStruct((n, 1), jnp.float32),
      jax.ShapeDtypeStruct((n, 1), jnp.float32),
  ]
  return pl.pallas_call(
      body,
      grid=(grid,),
      in_specs=[
          pl.BlockSpec((blk, d), lambda i: (i, 0)),
          pl.BlockSpec((blk, d), lambda i: (i, 0)),
          pl.BlockSpec((n, d), lambda i: (0, 0)),
          pl.BlockSpec((n, d), lambda i: (0, 0)),
      ],
      out_specs=[col, col, col, pl.BlockSpec((1, n), lambda i: (0, 0)),
                 col, col, col],
      out_shape=outs,
  )(an, bn, an, bn)


def _contrast(a, b, fcp):
  an = _mlp_norm(a, fcp)
  bn = _mlp_norm(b, fcp)
  ra, rb, sr, sc, dg, da, db = _contrast_sums(an, bn)
  ra, rb, sr, dg, da, db = (v[:, 0] for v in (ra, rb, sr, dg, da, db))
  sc = sc[0, :]
  l1 = -jnp.log(dg / (ra + sr - da))
  l2 = -jnp.log(dg / (rb + sc - db))
  return ((l1 + l2) * 0.5).mean()


def kernel(params, g_edge_index, t_edge_index, s_edge_index):
  x = params['all_embed']
  g_src, g_dst = _prep_edges(g_edge_index)
  t_src, t_dst = _prep_edges(t_edge_index)
  s_src, s_dst = _prep_edges(s_edge_index)

  invoke = _graphsage(x, g_src, g_dst, params['invoke'])
  tag = _graphsage(x, t_src, t_dst, params['tag'])
  sem = _graphsage(x, s_src, s_dst, params['sem'])

  invoke_m, invoke_a = invoke[:N_MASHUP], invoke[N_MASHUP:]
  tag_m, tag_a = tag[:N_MASHUP], tag[N_MASHUP:]
  sem_m, sem_a = sem[:N_MASHUP], sem[N_MASHUP:]

  m_loss1 = _contrast(tag_m, sem_m, params['fc1'])
  a_loss1 = _contrast(tag_a, sem_a, params['fc2'])
  m_emd1 = jnp.concatenate([tag_m, sem_m], axis=-1)
  a_emd1 = jnp.concatenate([tag_a, sem_a], axis=-1)
  m_loss2 = _contrast(m_emd1, invoke_m, params['fc3'])
  a_loss2 = _contrast(a_emd1, invoke_a, params['fc4'])

  m_emd = jnp.concatenate([m_emd1, invoke_m], axis=-1)
  a_emd = jnp.concatenate([a_emd1, invoke_a], axis=-1)
  node_emd = jnp.concatenate([m_emd, a_emd], axis=0)
  loss = 0.4 * (m_loss1 + a_loss1) + 0.6 * (m_loss2 + a_loss2)
  return node_emd, loss


# speedup vs baseline: 2.3391x; 1.0797x over previous
import functools

import jax
import jax.numpy as jnp
from jax import lax
from jax.experimental import pallas as pl
from jax.experimental.pallas import tpu as pltpu
from jax.experimental.pallas import tpu_sc as plsc

N_MASHUP = 5000
N_NODES = 10000
EMB = 128
LOCAL = 64
N_EDGES = 320000
TAU = 0.6

NC = 2
NS = 16
NW = NC * NS
CH = 128
EPT = 10240
KCH = EPT // CH
EPAD = NW * EPT
NNP = 10240
RPS = NNP // NS
DEG_W = 128


HKCH = KCH // 2


@functools.lru_cache(maxsize=None)
def _make_sc_agg(d):
  mesh = plsc.VectorSubcoreMesh(core_axis_name="c", subcore_axis_name="s")
  out_type = [jax.ShapeDtypeStruct((NC, NNP, d), jnp.float32)]
  scratch = [
      pltpu.VMEM((HKCH, CH), jnp.int32),
      pltpu.VMEM((HKCH, CH), jnp.int32),
      pltpu.VMEM((CH, d), jnp.float32),
      pltpu.VMEM((CH, d), jnp.float32),
      pltpu.VMEM_SHARED((NNP, d), jnp.float32),
      pltpu.SemaphoreType.DMA,
      pltpu.SemaphoreType.DMA,
      pltpu.SemaphoreType.DMA,
      pltpu.SemaphoreType.DMA,
  ]

  def body(x_hbm, src_hbm, dst_hbm, zeros_hbm, out_hbm,
           src_v, dst_v, rows0, rows1, acc_sh, g0, g1, s0, s1):
    ci = lax.axis_index("c")
    si = lax.axis_index("s")
    wid = si * NC + ci
    pltpu.sync_copy(zeros_hbm.at[pl.ds(si * RPS, RPS)],
                    acc_sh.at[pl.ds(si * RPS, RPS)])
    plsc.subcore_barrier()

    rows = (rows0, rows1)
    gsem = (g0, g1)
    ssem = (s0, s1)

    def gath(c, b):
      pltpu.async_copy(x_hbm.at[src_v.at[c]], rows[b], gsem[b])

    def gath_wait(b):
      pltpu.make_async_copy(x_hbm.at[src_v.at[0]], rows[b], gsem[b]).wait()

    def scat(c, b):
      pltpu.async_copy(rows[b], acc_sh.at[dst_v.at[c]], ssem[b], add=True)

    def scat_wait(b):
      pltpu.make_async_copy(rows[b], acc_sh.at[dst_v.at[0]], ssem[b]).wait()

    for half in range(2):
      pltpu.sync_copy(src_hbm.at[wid, pl.ds(half * HKCH, HKCH)], src_v)
      pltpu.sync_copy(dst_hbm.at[wid, pl.ds(half * HKCH, HKCH)], dst_v)
      gath(0, 0)
      gath(1, 1)

      @pl.loop(0, HKCH // 2 - 1)
      def _(jj):
        c = 2 * jj
        gath_wait(0)
        scat(c, 0)
        gath_wait(1)
        scat(c + 1, 1)
        scat_wait(0)
        gath(c + 2, 0)
        scat_wait(1)
        gath(c + 3, 1)

      gath_wait(0)
      scat(HKCH - 2, 0)
      gath_wait(1)
      scat(HKCH - 1, 1)
      scat_wait(0)
      scat_wait(1)

    plsc.subcore_barrier()
    pltpu.sync_copy(acc_sh.at[pl.ds(si * RPS, RPS)],
                    out_hbm.at[ci, pl.ds(si * RPS, RPS)])

  return pl.kernel(body, out_type=out_type, mesh=mesh,
                   scratch_types=scratch, name=f"sc_agg_d{d}")


@functools.lru_cache(maxsize=None)
def _make_sc_deg():
  mesh = plsc.VectorSubcoreMesh(core_axis_name="c", subcore_axis_name="s")
  out_type = [jax.ShapeDtypeStruct((NC, NNP, DEG_W), jnp.float32)]
  scratch = [
      pltpu.VMEM((KCH, CH), jnp.int32),
      pltpu.VMEM((CH, DEG_W), jnp.float32),
      pltpu.VMEM_SHARED((NNP, DEG_W), jnp.float32),
  ]

  def body(dst_hbm, zdeg_hbm, ones_hbm, deg_hbm, dst_v, ones_v, deg_sh):
    ci = lax.axis_index("c")
    si = lax.axis_index("s")
    wid = si * NC + ci
    pltpu.sync_copy(dst_hbm.at[wid], dst_v)
    pltpu.sync_copy(ones_hbm, ones_v)
    pltpu.sync_copy(zdeg_hbm.at[pl.ds(si * RPS, RPS)],
                    deg_sh.at[pl.ds(si * RPS, RPS)])
    plsc.subcore_barrier()

    @pl.loop(0, KCH)
    def _(j):
      pltpu.sync_copy(ones_v, deg_sh.at[dst_v.at[j]], add=True)

    plsc.subcore_barrier()
    pltpu.sync_copy(deg_sh.at[pl.ds(si * RPS, RPS)],
                    deg_hbm.at[ci, pl.ds(si * RPS, RPS)])

  return pl.kernel(body, out_type=out_type, mesh=mesh,
                   scratch_types=scratch, name="sc_deg")


def _prep_edges(edge_index):
  src = edge_index[0]
  dst = edge_index[1]
  src = jnp.concatenate([src, jnp.zeros((EPAD - N_EDGES,), jnp.int32)])
  dst = jnp.concatenate(
      [dst, jnp.full((EPAD - N_EDGES,), N_NODES, jnp.int32)])
  return src.reshape(NW, KCH, CH), dst.reshape(NW, KCH, CH)


def _sc_agg(x, src3, dst3):
  d = x.shape[1]
  zeros = jnp.zeros((NNP, d), jnp.float32)
  return _make_sc_agg(d)(x, src3, dst3, zeros)[0]


def _sc_deg(dst3):
  zdeg = jnp.zeros((NNP, DEG_W), jnp.float32)
  ones = jnp.ones((CH, DEG_W), jnp.float32)
  return _make_sc_deg()(dst3, zdeg, ones)[0]


def _sage_dense(x, agg2, deg2, wsn, b, relu):
  n, din = x.shape
  dout = wsn.shape[1]
  blk = 2000
  grid = n // blk

  def body(x_ref, a_ref, d_ref, w_ref, b_ref, o_ref):
    agg = a_ref[0] + a_ref[1]
    deg = d_ref[0, :, 0:1] + d_ref[1, :, 0:1]
    hn = agg / jnp.maximum(deg, 1.0)
    xa = jnp.concatenate([x_ref[...], hn], axis=1)
    o = jnp.dot(xa, w_ref[...], preferred_element_type=jnp.float32)
    o = o + b_ref[...]
    if relu:
      o = jnp.maximum(o, 0.0)
    o_ref[...] = o

  return pl.pallas_call(
      body,
      grid=(grid,),
      in_specs=[
          pl.BlockSpec((blk, din), lambda i: (i, 0)),
          pl.BlockSpec((NC, blk, din), lambda i: (0, i, 0)),
          pl.BlockSpec((NC, blk, DEG_W), lambda i: (0, i, 0)),
          pl.BlockSpec((2 * din, dout), lambda i: (0, 0)),
          pl.BlockSpec((1, dout), lambda i: (0, 0)),
      ],
      out_specs=pl.BlockSpec((blk, dout), lambda i: (i, 0)),
      out_shape=jax.ShapeDtypeStruct((n, dout), jnp.float32),
  )(x, agg2, deg2, wsn, b)


def _graphsage(x, src3, dst3, p):
  dh = p['W1s'].shape[1]
  deg = _sc_deg(dst3)
  w1 = jnp.concatenate([p['W1s'], p['W1n']], axis=0)
  b1 = p['b1'].reshape(1, dh)
  if dh < EMB:
    w1 = jnp.pad(w1, ((0, 0), (0, EMB - dh)))
    b1 = jnp.pad(b1, ((0, 0), (0, EMB - dh)))
  agg1 = _sc_agg(x, src3, dst3)
  h = _sage_dense(x, agg1, deg, w1, b1, True)
  agg2 = _sc_agg(h, src3, dst3)
  w2s, w2n = p['W2s'], p['W2n']
  if dh < EMB:
    w2s = jnp.pad(w2s, ((0, EMB - dh), (0, 0)))
    w2n = jnp.pad(w2n, ((0, EMB - dh), (0, 0)))
  w2 = jnp.concatenate([w2s, w2n], axis=0)
  return _sage_dense(h, agg2, deg, w2, p['b2'].reshape(1, dh), False)


def _mlp_norm(a, fcp):
  n, d = a.shape
  blk = 1000
  grid = n // blk

  def body(a_ref, w1, b1, w2, b2, o_ref):
    h = jnp.dot(a_ref[...], w1[...], preferred_element_type=jnp.float32)
    h = jnp.maximum(h + b1[...], 0.0)
    z = jnp.dot(h, w2[...], preferred_element_type=jnp.float32) + b2[...]
    nrm = jnp.sqrt(jnp.sum(z * z, axis=1, keepdims=True))
    o_ref[...] = z / jnp.maximum(nrm, 1e-12)

  return pl.pallas_call(
      body,
      grid=(grid,),
      in_specs=[
          pl.BlockSpec((blk, d), lambda i: (i, 0)),
          pl.BlockSpec((d, d), lambda i: (0, 0)),
          pl.BlockSpec((1, d), lambda i: (0, 0)),
          pl.BlockSpec((d, d), lambda i: (0, 0)),
          pl.BlockSpec((1, d), lambda i: (0, 0)),
      ],
      out_specs=pl.BlockSpec((blk, d), lambda i: (i, 0)),
      out_shape=jax.ShapeDtypeStruct((n, d), jnp.float32),
  )(a, fcp['W1'], fcp['b1'].reshape(1, d), fcp['W2'], fcp['b2'].reshape(1, d))


def _contrast_sums(an, bn):
  n, d = an.shape
  blk = 200
  grid = n // blk
  inv_tau = 1.0 / TAU

  def body(ai_ref, bi_ref, an_ref, bn_ref,
           ra_ref, rb_ref, sr_ref, sc_ref, dg_ref, da_ref, db_ref):
    i = pl.program_id(0)
    ai = ai_ref[...]
    bi = bi_ref[...]
    dn = (((1,), (1,)), ((), ()))
    eaa = jnp.exp(lax.dot_general(ai, an_ref[...], dn,
                                  preferred_element_type=jnp.float32)
                  * inv_tau)
    ra_ref[...] = jnp.sum(eaa, axis=1, keepdims=True)
    ebb = jnp.exp(lax.dot_general(bi, bn_ref[...], dn,
                                  preferred_element_type=jnp.float32)
                  * inv_tau)
    rb_ref[...] = jnp.sum(ebb, axis=1, keepdims=True)
    eab = jnp.exp(lax.dot_general(ai, bn_ref[...], dn,
                                  preferred_element_type=jnp.float32)
                  * inv_tau)
    sr_ref[...] = jnp.sum(eab, axis=1, keepdims=True)
    colpart = jnp.sum(eab, axis=0, keepdims=True)

    @pl.when(i == 0)
    def _():
      sc_ref[...] = colpart

    @pl.when(i > 0)
    def _():
      sc_ref[...] += colpart

    dg_ref[...] = jnp.exp(jnp.sum(ai * bi, axis=1, keepdims=True) * inv_tau)
    da_ref[...] = jnp.exp(jnp.sum(ai * ai, axis=1, keepdims=True) * inv_tau)
    db_ref[...] = jnp.exp(jnp.sum(bi * bi, axis=1, keepdims=True) * inv_tau)

  col = pl.BlockSpec((blk, 1), lambda i: (i, 0))
  outs = [
      jax.ShapeDtypeStruct((n, 1), jnp.float32),
      jax.ShapeDtypeStruct((n, 1), jnp.float32),
      jax.ShapeDtypeStruct((n, 1), jnp.float32),
      jax.ShapeDtypeStruct((1, n), jnp.float32),
      jax.ShapeDtypeStruct((n, 1), jnp.float32),
      jax.ShapeDtypeStruct((n, 1), jnp.float32),
      jax.ShapeDtypeStruct((n, 1), jnp.float32),
  ]
  return pl.pallas_call(
      body,
      grid=(grid,),
      in_specs=[
          pl.BlockSpec((blk, d), lambda i: (i, 0)),
          pl.BlockSpec((blk, d), lambda i: (i, 0)),
          pl.BlockSpec((n, d), lambda i: (0, 0)),
          pl.BlockSpec((n, d), lambda i: (0, 0)),
      ],
      out_specs=[col, col, col, pl.BlockSpec((1, n), lambda i: (0, 0)),
                 col, col, col],
      out_shape=outs,
  )(an, bn, an, bn)


def _contrast(a, b, fcp):
  an = _mlp_norm(a, fcp)
  bn = _mlp_norm(b, fcp)
  ra, rb, sr, sc, dg, da, db = _contrast_sums(an, bn)
  ra, rb, sr, dg, da, db = (v[:, 0] for v in (ra, rb, sr, dg, da, db))
  sc = sc[0, :]
  l1 = -jnp.log(dg / (ra + sr - da))
  l2 = -jnp.log(dg / (rb + sc - db))
  return ((l1 + l2) * 0.5).mean()


def kernel(params, g_edge_index, t_edge_index, s_edge_index):
  x = params['all_embed']
  g_src, g_dst = _prep_edges(g_edge_index)
  t_src, t_dst = _prep_edges(t_edge_index)
  s_src, s_dst = _prep_edges(s_edge_index)

  invoke = _graphsage(x, g_src, g_dst, params['invoke'])
  tag = _graphsage(x, t_src, t_dst, params['tag'])
  sem = _graphsage(x, s_src, s_dst, params['sem'])

  invoke_m, invoke_a = invoke[:N_MASHUP], invoke[N_MASHUP:]
  tag_m, tag_a = tag[:N_MASHUP], tag[N_MASHUP:]
  sem_m, sem_a = sem[:N_MASHUP], sem[N_MASHUP:]

  m_loss1 = _contrast(tag_m, sem_m, params['fc1'])
  a_loss1 = _contrast(tag_a, sem_a, params['fc2'])
  m_emd1 = jnp.concatenate([tag_m, sem_m], axis=-1)
  a_emd1 = jnp.concatenate([tag_a, sem_a], axis=-1)
  m_loss2 = _contrast(m_emd1, invoke_m, params['fc3'])
  a_loss2 = _contrast(a_emd1, invoke_a, params['fc4'])

  m_emd = jnp.concatenate([m_emd1, invoke_m], axis=-1)
  a_emd = jnp.concatenate([a_emd1, invoke_a], axis=-1)
  node_emd = jnp.concatenate([m_emd, a_emd], axis=0)
  loss = 0.4 * (m_loss1 + a_loss1) + 0.6 * (m_loss2 + a_loss2)
  return node_emd, loss
